# baseline (speedup 1.0000x reference)
import functools

import jax
import jax.numpy as jnp
from jax import lax
from jax.experimental import pallas as pl
from jax.experimental.pallas import tpu as pltpu
from jax.experimental.pallas import tpu_sc as plsc

N = 10000
E = 320000
D = 128

NC = 2
NS = 16
NW = NC * NS
EPW = E // NW
CH = 100
NCH = EPW // CH
RPT = N // NS
ZR = 125
BR = 1000
GRID = N // BR
BR1 = 1024
GRID1 = 10

DEGP = 10240

_mesh = plsc.VectorSubcoreMesh(core_axis_name="c", subcore_axis_name="s")


@functools.partial(
    pl.kernel,
    out_type=jax.ShapeDtypeStruct((NW, DEGP), jnp.float32),
    mesh=_mesh,
    compiler_params=pltpu.CompilerParams(needs_layout_passes=False, use_tc_tiling_on_sc=False),
    scratch_types=[
        pltpu.VMEM((DEGP,), jnp.float32),
        pltpu.VMEM((EPW,), jnp.int32),
    ],
)
def _deg_kernel(dst_hbm, out_hbm, dv, dbuf):
    cid = lax.axis_index("c")
    sid = lax.axis_index("s")
    wid = cid * NS + sid

    def zero_body(i, carry):
        dv[pl.ds(i * 16, 16)] = jnp.zeros((16,), jnp.float32)
        return carry

    lax.fori_loop(0, DEGP // 16, zero_body, 0)

    base = pl.multiple_of(wid * EPW, 8)
    pltpu.sync_copy(dst_hbm.at[pl.ds(base, EPW)], dbuf)

    ones = jnp.ones((16,), jnp.float32)

    def acc_body(j, carry):
        idx = dbuf[pl.ds(j * 16, 16)]
        plsc.addupdate_scatter(dv, [idx], ones)
        return carry

    lax.fori_loop(0, EPW // 16, acc_body, 0)
    pltpu.sync_copy(dv, out_hbm.at[wid])


@functools.partial(
    pl.kernel,
    out_type=jax.ShapeDtypeStruct((NC, N, D), jnp.float32),
    mesh=_mesh,
    compiler_params=pltpu.CompilerParams(needs_layout_passes=False, use_tc_tiling_on_sc=False),
    scratch_types=[
        pltpu.VMEM_SHARED((N, D), jnp.float32),
        pltpu.VMEM((NCH, CH), jnp.int32),
        pltpu.VMEM((NCH, CH), jnp.int32),
        pltpu.VMEM((CH, D), jnp.float32),
        pltpu.VMEM((CH, D), jnp.float32),
        pltpu.SemaphoreType.DMA,
        pltpu.SemaphoreType.DMA,
        pltpu.SemaphoreType.DMA,
    ],
)
def _scatter_kernel(g_hbm, src_hbm, dst_hbm, out_hbm,
                    acc, sbuf, dbuf, rb0, rb1, sem0, sem1, six):
    cid = lax.axis_index("c")
    sid = lax.axis_index("s")
    wid = cid * NS + sid

    ix_s = pltpu.async_copy(src_hbm.at[pl.ds(wid * NCH, NCH)], sbuf, six)
    ix_d = pltpu.async_copy(dst_hbm.at[pl.ds(wid * NCH, NCH)], dbuf, six)

    def zero_rows(i, carry):
        def zrow(j, inner):
            rb0[i, pl.ds(j * 16, 16)] = jnp.zeros((16,), jnp.float32)
            return inner
        return lax.fori_loop(0, D // 16, zrow, carry)

    lax.fori_loop(0, CH, zero_rows, 0)

    for k in range(RPT // CH):
        pltpu.sync_copy(rb0, acc.at[pl.ds(sid * RPT + k * CH, CH)])
    rem = RPT - (RPT // CH) * CH
    if rem:
        pltpu.sync_copy(rb0.at[pl.ds(0, rem)],
                        acc.at[pl.ds(sid * RPT + (RPT // CH) * CH, rem)])
    ix_s.wait()
    ix_d.wait()
    plsc.subcore_barrier()

    pltpu.async_copy(g_hbm.at[sbuf.at[0]], rb0, sem0)

    def pair_body(p, carry):
        k0 = 2 * p
        k1 = k0 + 1
        k2 = k0 + 2
        pltpu.async_copy(g_hbm.at[sbuf.at[k1]], rb1, sem1)
        pltpu.make_async_copy(g_hbm.at[sbuf.at[k0]], rb0, sem0).wait()
        pltpu.sync_copy(rb0, acc.at[dbuf.at[k0]], add=True)
        pltpu.async_copy(g_hbm.at[sbuf.at[k2]], rb0, sem0)
        pltpu.make_async_copy(g_hbm.at[sbuf.at[k1]], rb1, sem1).wait()
        pltpu.sync_copy(rb1, acc.at[dbuf.at[k1]], add=True)
        return carry

    lax.fori_loop(0, (NCH - 1) // 2, pair_body, 0)
    if NCH % 2 == 0:
        pltpu.async_copy(g_hbm.at[sbuf.at[NCH - 1]], rb1, sem1)
        pltpu.make_async_copy(g_hbm.at[sbuf.at[NCH - 2]], rb0, sem0).wait()
        pltpu.sync_copy(rb0, acc.at[dbuf.at[NCH - 2]], add=True)
        pltpu.make_async_copy(g_hbm.at[sbuf.at[NCH - 1]], rb1, sem1).wait()
        pltpu.sync_copy(rb1, acc.at[dbuf.at[NCH - 1]], add=True)
    else:
        pltpu.make_async_copy(g_hbm.at[sbuf.at[NCH - 1]], rb0, sem0).wait()
        pltpu.sync_copy(rb0, acc.at[dbuf.at[NCH - 1]], add=True)

    plsc.subcore_barrier()
    pltpu.sync_copy(acc.at[pl.ds(sid * RPT, RPT)],
                    out_hbm.at[cid, pl.ds(sid * RPT, RPT), :])


def _in_body(x_ref, wi_ref, bi_ref, w0_ref, dp_ref, h_ref, g_ref, dc_ref):
    cd = (((1,), (1,)), ((), ()))
    h = lax.dot_general(x_ref[...], wi_ref[...], cd,
                        preferred_element_type=jnp.float32) + bi_ref[...]
    h_ref[...] = h
    hw = lax.dot_general(h, w0_ref[...], cd,
                         preferred_element_type=jnp.float32)
    ones = jnp.ones((NW, 1), jnp.float32)
    s = lax.dot_general(dp_ref[...], ones, (((0,), (0,)), ((), ())),
                        preferred_element_type=jnp.float32)
    dinv = lax.rsqrt(1.0 + s)
    dc_ref[...] = dinv
    g_ref[...] = hw * dinv


def _mid_body(h_ref, g_ref, pa_ref, pb_ref, dinv_ref, cb_ref, lg_ref, lb_ref,
              w1_ref, h1_ref, g1_ref):
    dinv = dinv_ref[...]
    y = dinv * (pa_ref[0] + pb_ref[0] + g_ref[...]) + cb_ref[...]
    m = jnp.mean(y, axis=1, keepdims=True)
    c = y - m
    v = jnp.mean(c * c, axis=1, keepdims=True)
    yn = c * lax.rsqrt(v + 1e-5) * lg_ref[...] + lb_ref[...]
    h1 = h_ref[...] + jnp.maximum(yn, 0.0)
    h1_ref[...] = h1
    cd = (((1,), (1,)), ((), ()))
    g1_ref[...] = lax.dot_general(h1, w1_ref[...], cd,
                                  preferred_element_type=jnp.float32) * dinv


def _out_body(h_ref, g_ref, pa_ref, pb_ref, dinv_ref, cb_ref, lg_ref, lb_ref,
              ow_ref, ob_ref, o_ref):
    dinv = dinv_ref[...]
    y = dinv * (pa_ref[0] + pb_ref[0] + g_ref[...]) + cb_ref[...]
    m = jnp.mean(y, axis=1, keepdims=True)
    c = y - m
    v = jnp.mean(c * c, axis=1, keepdims=True)
    yn = c * lax.rsqrt(v + 1e-5) * lg_ref[...] + lb_ref[...]
    h2 = h_ref[...] + jnp.maximum(yn, 0.0)
    cd = (((1,), (1,)), ((), ()))
    o_ref[...] = lax.dot_general(h2, ow_ref[...], cd,
                                 preferred_element_type=jnp.float32) + ob_ref[...]


def _row_spec():
    return pl.BlockSpec((BR, D), lambda i: (i, 0))


def _w_spec():
    return pl.BlockSpec((D, D), lambda i: (0, 0))


def _v_spec():
    return pl.BlockSpec((1, D), lambda i: (0, 0))


def _dinv_spec():
    return pl.BlockSpec((BR, 1), lambda i: (i, 0))


def _p_spec(c):
    return pl.BlockSpec((1, BR, D), lambda i, c=c: (c, i, 0))


_f32 = jnp.float32


def kernel(x, edge_index, in_w, in_b, conv_w0, conv_b0, ln_g0, ln_b0,
           conv_w1, conv_b1, ln_g1, ln_b1, out_w, out_b):
    src2d = edge_index[0].reshape(E // CH, CH)
    dst2d = edge_index[1].reshape(E // CH, CH)
    dst_flat = edge_index[1]

    degp = _deg_kernel(dst_flat)

    h, g0, dinv_col = pl.pallas_call(
        _in_body,
        grid=(GRID1,),
        in_specs=[pl.BlockSpec((BR1, D), lambda i: (i, 0)),
                  pl.BlockSpec((D, D), lambda i: (0, 0)),
                  pl.BlockSpec((1, D), lambda i: (0, 0)),
                  pl.BlockSpec((D, D), lambda i: (0, 0)),
                  pl.BlockSpec((NW, BR1), lambda i: (0, i))],
        out_specs=[pl.BlockSpec((BR1, D), lambda i: (i, 0)),
                   pl.BlockSpec((BR1, D), lambda i: (i, 0)),
                   pl.BlockSpec((BR1, 1), lambda i: (i, 0))],
        out_shape=[jax.ShapeDtypeStruct((N, D), _f32),
                   jax.ShapeDtypeStruct((N, D), _f32),
                   jax.ShapeDtypeStruct((N, 1), _f32)],
    )(x, in_w, in_b.reshape(1, D), conv_w0, degp)

    p0 = _scatter_kernel(g0, src2d, dst2d)

    h1, g1 = pl.pallas_call(
        _mid_body,
        grid=(GRID,),
        in_specs=[_row_spec(), _row_spec(), _p_spec(0), _p_spec(1),
                  _dinv_spec(), _v_spec(), _v_spec(), _v_spec(), _w_spec()],
        out_specs=[_row_spec(), _row_spec()],
        out_shape=[jax.ShapeDtypeStruct((N, D), _f32),
                   jax.ShapeDtypeStruct((N, D), _f32)],
    )(h, g0, p0, p0, dinv_col, conv_b0.reshape(1, D), ln_g0.reshape(1, D),
      ln_b0.reshape(1, D), conv_w1)

    p1 = _scatter_kernel(g1, src2d, dst2d)

    out = pl.pallas_call(
        _out_body,
        grid=(GRID,),
        in_specs=[_row_spec(), _row_spec(), _p_spec(0), _p_spec(1),
                  _dinv_spec(), _v_spec(), _v_spec(), _v_spec(), _w_spec(),
                  _v_spec()],
        out_specs=_row_spec(),
        out_shape=jax.ShapeDtypeStruct((N, D), _f32),
    )(h1, g1, p1, p1, dinv_col, conv_b1.reshape(1, D), ln_g1.reshape(1, D),
      ln_b1.reshape(1, D), out_w, out_b.reshape(1, D))

    return out
